# speedup vs baseline: 3.0740x; 3.0740x over previous
import jax
import jax.numpy as jnp
from jax.experimental import pallas as pl
from jax.experimental.pallas import tpu as pltpu

_B, _N, _D, _H, _E = 2, 10000, 128, 256, 10
_TM = 2000


def _moe_tile(x_ref, Wr1_ref, br1_ref, Wr2_ref, br2_ref,
              W1_ref, b1_ref, W2_ref, b2_ref, W3_ref, b3_ref, out_ref):
    x = x_ref[...]
    h = jnp.maximum(
        jnp.dot(x, Wr1_ref[...], preferred_element_type=jnp.float32)
        + br1_ref[...][None, :], 0.0)
    logits = (jnp.dot(h, Wr2_ref[...], preferred_element_type=jnp.float32)
              + br2_ref[...][None, :])
    ids = jax.lax.broadcasted_iota(jnp.int32, logits.shape, 1)
    l1 = jnp.max(logits, axis=-1, keepdims=True)
    a1 = jnp.min(jnp.where(logits == l1, ids, _E), axis=-1, keepdims=True)
    masked = jnp.where(ids == a1, -jnp.inf, logits)
    l2 = jnp.max(masked, axis=-1, keepdims=True)
    a2 = jnp.min(jnp.where(masked == l2, ids, _E), axis=-1, keepdims=True)
    ed = jnp.exp(l2 - l1)
    g1 = 1.0 / (1.0 + ed)
    g2 = ed / (1.0 + ed)

    acc = jnp.zeros((x.shape[0], _D), dtype=jnp.float32)
    for e in range(_E):
        ge = (jnp.where(a1 == e, g1, 0.0) + jnp.where(a2 == e, g2, 0.0))
        h1 = jnp.maximum(
            jnp.dot(x, W1_ref[e], preferred_element_type=jnp.float32)
            + b1_ref[e][None, :], 0.0)
        h2 = jnp.maximum(
            jnp.dot(h1, W2_ref[e], preferred_element_type=jnp.float32)
            + b2_ref[e][None, :], 0.0)
        o = (jnp.dot(h2, W3_ref[e], preferred_element_type=jnp.float32)
             + b3_ref[e][None, :])
        acc = acc + ge * o
    out_ref[...] = acc


def kernel(x, Wr1, br1, Wr2, br2, W1, b1, W2, b2, W3, b3):
    M = _B * _N
    xf = x.reshape(M, _D)
    full = lambda shape: pl.BlockSpec(shape, lambda i: (0,) * len(shape))
    out = pl.pallas_call(
        _moe_tile,
        grid=(M // _TM,),
        in_specs=[
            pl.BlockSpec((_TM, _D), lambda i: (i, 0)),
            full((_D, 128)), full((128,)), full((128, _E)), full((_E,)),
            full((_E, _D, _H)), full((_E, _H)),
            full((_E, _H, _H)), full((_E, _H)),
            full((_E, _H, _D)), full((_E, _D)),
        ],
        out_specs=pl.BlockSpec((_TM, _D), lambda i: (i, 0)),
        out_shape=jax.ShapeDtypeStruct((M, _D), jnp.float32),
    )(xf, Wr1, br1, Wr2, br2, W1, b1, W2, b2, W3, b3)
    return out.reshape(_B, _N, _D)
